# baseline (speedup 1.0000x reference)
import functools

import jax
import jax.numpy as jnp
from jax import lax
from jax.experimental import pallas as pl
from jax.experimental.pallas import tpu as pltpu, tpu_sc as plsc

NC = 2
NS = 16
L = 16
NW = NC * NS

E = 320000
D = 128
PER_W = E // NW
C = 80
NCHUNK = PER_W // C
NGROUP = C // L

_mesh = plsc.VectorSubcoreMesh(
    core_axis_name="c", subcore_axis_name="s", num_cores=NC, num_subcores=NS
)


@functools.partial(
    pl.kernel,
    out_type=jax.ShapeDtypeStruct((E,), jnp.float32),
    mesh=_mesh,
    compiler_params=pltpu.CompilerParams(needs_layout_passes=False),
    scratch_types=[
        pltpu.VMEM((C,), jnp.int32),
        pltpu.VMEM((C,), jnp.int32),
        pltpu.VMEM((C, D), jnp.float32),
        pltpu.VMEM((C, D), jnp.float32),
        pltpu.VMEM((C,), jnp.float32),
        pltpu.SemaphoreType.DMA,
        pltpu.SemaphoreType.DMA,
    ],
)
def _sc_dot_kernel(emb_hbm, idx0_hbm, idx1_hbm, out_hbm,
                   idx0_v, idx1_v, rows0_v, rows1_v, out_v, sem0, sem1):
    wid = lax.axis_index("s") * NC + lax.axis_index("c")
    base = wid * PER_W
    lane = lax.iota(jnp.int32, L)

    def chunk_body(ci, _):
        cbase = base + ci * C
        pltpu.sync_copy(idx0_hbm.at[pl.ds(cbase, C)], idx0_v)
        pltpu.sync_copy(idx1_hbm.at[pl.ds(cbase, C)], idx1_v)
        cp0 = pltpu.async_copy(emb_hbm.at[idx0_v], rows0_v, sem0)
        cp1 = pltpu.async_copy(emb_hbm.at[idx1_v], rows1_v, sem1)
        cp0.wait()
        cp1.wait()

        def group_body(g, _):
            row_ids = g * L + lane
            accs = [jnp.zeros((L,), jnp.float32) for _ in range(4)]
            for d in range(D):
                col = jnp.full((L,), d, jnp.int32)
                a = plsc.load_gather(rows0_v, [row_ids, col])
                b = plsc.load_gather(rows1_v, [row_ids, col])
                accs[d % 4] = accs[d % 4] + a * b
            out_v[pl.ds(g * L, L)] = (accs[0] + accs[1]) + (accs[2] + accs[3])
            return 0

        lax.fori_loop(0, NGROUP, group_body, 0)
        pltpu.sync_copy(out_v, out_hbm.at[pl.ds(cbase, C)])
        return 0

    lax.fori_loop(0, NCHUNK, chunk_body, 0)


def kernel(node_embeddings, edge_index):
    idx = edge_index.astype(jnp.int32)
    return _sc_dot_kernel(node_embeddings, idx[0], idx[1])


# speedup vs baseline: 1.1727x; 1.1727x over previous
import functools

import jax
import jax.numpy as jnp
from jax import lax
from jax.experimental import pallas as pl
from jax.experimental.pallas import tpu as pltpu, tpu_sc as plsc

NC = 2
NS = 16
L = 16
NW = NC * NS

E = 320000
D = 128
PER_W = E // NW
C = 80
NCHUNK = PER_W // C
NGROUP = C // L

_mesh = plsc.VectorSubcoreMesh(
    core_axis_name="c", subcore_axis_name="s", num_cores=NC, num_subcores=NS
)


@functools.partial(
    pl.kernel,
    out_type=jax.ShapeDtypeStruct((E,), jnp.float32),
    mesh=_mesh,
    compiler_params=pltpu.CompilerParams(needs_layout_passes=False),
    scratch_types=[
        pltpu.VMEM((NCHUNK, C), jnp.int32),
        pltpu.VMEM((NCHUNK, C), jnp.int32),
        pltpu.VMEM((2, C, D), jnp.float32),
        pltpu.VMEM((2, C, D), jnp.float32),
        pltpu.VMEM((2, C), jnp.float32),
        pltpu.SemaphoreType.DMA,
        pltpu.SemaphoreType.DMA,
        pltpu.SemaphoreType.DMA,
        pltpu.SemaphoreType.DMA,
        pltpu.SemaphoreType.DMA,
        pltpu.SemaphoreType.DMA,
    ],
)
def _sc_dot_kernel(emb_hbm, idx0_hbm, idx1_hbm, out_hbm,
                   idx0_v, idx1_v, rows0_v, rows1_v, out_v,
                   sem_r0a, sem_r0b, sem_r1a, sem_r1b, sem_oa, sem_ob):
    wid = lax.axis_index("s") * NC + lax.axis_index("c")
    base = wid * PER_W
    lane = lax.iota(jnp.int32, L)
    sem_r0 = (sem_r0a, sem_r0b)
    sem_r1 = (sem_r1a, sem_r1b)
    sem_o = (sem_oa, sem_ob)

    pltpu.sync_copy(idx0_hbm.at[wid], idx0_v)
    pltpu.sync_copy(idx1_hbm.at[wid], idx1_v)

    def start_gather(ci, p):
        pltpu.async_copy(
            emb_hbm.at[idx0_v.at[ci]], rows0_v.at[p], sem_r0[p])
        pltpu.async_copy(
            emb_hbm.at[idx1_v.at[ci]], rows1_v.at[p], sem_r1[p])

    def wait_gather(p):
        pltpu.make_async_copy(emb_hbm.at[idx0_v.at[0]],
                              rows0_v.at[p], sem_r0[p]).wait()
        pltpu.make_async_copy(emb_hbm.at[idx1_v.at[0]],
                              rows1_v.at[p], sem_r1[p]).wait()

    def compute(ci, p):
        rows0 = rows0_v.at[p]
        rows1 = rows1_v.at[p]

        def group_body(g, _):
            row_ids = g * L + lane
            accs = [jnp.zeros((L,), jnp.float32) for _ in range(4)]
            for d in range(D):
                col = jnp.full((L,), d, jnp.int32)
                a = plsc.load_gather(rows0, [row_ids, col])
                b = plsc.load_gather(rows1, [row_ids, col])
                accs[d % 4] = accs[d % 4] + a * b
            out_v[p, pl.ds(g * L, L)] = (accs[0] + accs[1]) + (accs[2] + accs[3])
            return 0

        lax.fori_loop(0, NGROUP, group_body, 0)
        pltpu.async_copy(out_v.at[p], out_hbm.at[pl.ds(base + ci * C, C)],
                         sem_o[p])

    start_gather(0, 0)

    def chunk_pair(i, _):
        c0 = i * 2
        start_gather(c0 + 1, 1)
        wait_gather(0)
        compute(c0, 0)
        nxt = jnp.minimum(c0 + 2, NCHUNK - 1)
        start_gather(nxt, 0)
        wait_gather(1)
        compute(c0 + 1, 1)
        return 0

    def chunk_pair_guarded(i, _):
        @pl.when(i > 0)
        def _():
            pltpu.make_async_copy(out_v.at[0], out_hbm.at[pl.ds(base, C)],
                                  sem_o[0]).wait()
            pltpu.make_async_copy(out_v.at[1], out_hbm.at[pl.ds(base, C)],
                                  sem_o[1]).wait()
        chunk_pair(i, None)
        return 0

    lax.fori_loop(0, NCHUNK // 2, chunk_pair_guarded, 0)
    pltpu.make_async_copy(out_v.at[0], out_hbm.at[pl.ds(base, C)],
                          sem_o[0]).wait()
    pltpu.make_async_copy(out_v.at[1], out_hbm.at[pl.ds(base, C)],
                          sem_o[1]).wait()
    wait_gather(0)
    compute(NCHUNK - 1, 0)
    pltpu.make_async_copy(out_v.at[0], out_hbm.at[pl.ds(base, C)],
                          sem_o[0]).wait()


def kernel(node_embeddings, edge_index):
    idx = edge_index.astype(jnp.int32).reshape(2, NW, NCHUNK, C)
    return _sc_dot_kernel(node_embeddings, idx[0], idx[1])


# speedup vs baseline: 5.4315x; 4.6315x over previous
import functools

import jax
import jax.numpy as jnp
from jax import lax
from jax.experimental import pallas as pl
from jax.experimental.pallas import tpu as pltpu, tpu_sc as plsc

NC = 2
NS = 16
L = 16
NW = NC * NS

E = 320000
D = 128
PER_W = E // NW
C = 80
NCHUNK = PER_W // C
NGROUP = C // L

_mesh = plsc.VectorSubcoreMesh(
    core_axis_name="c", subcore_axis_name="s", num_cores=NC, num_subcores=NS
)


@functools.partial(
    pl.kernel,
    out_type=jax.ShapeDtypeStruct((E,), jnp.float32),
    mesh=_mesh,
    compiler_params=pltpu.CompilerParams(needs_layout_passes=False),
    scratch_types=[
        pltpu.VMEM((NCHUNK, C), jnp.int32),
        pltpu.VMEM((NCHUNK, C), jnp.int32),
        pltpu.VMEM((2, C, D), jnp.float32),
        pltpu.VMEM((2, C, D), jnp.float32),
        pltpu.VMEM((2, C), jnp.float32),
        pltpu.SemaphoreType.DMA,
        pltpu.SemaphoreType.DMA,
        pltpu.SemaphoreType.DMA,
        pltpu.SemaphoreType.DMA,
        pltpu.SemaphoreType.DMA,
        pltpu.SemaphoreType.DMA,
    ],
)
def _sc_dot_kernel(emb_hbm, idx0_hbm, idx1_hbm, out_hbm,
                   idx0_v, idx1_v, rows0_v, rows1_v, out_v,
                   sem_r0a, sem_r0b, sem_r1a, sem_r1b, sem_oa, sem_ob):
    wid = lax.axis_index("s") * NC + lax.axis_index("c")
    base = wid * PER_W
    lane = lax.iota(jnp.int32, L)
    sem_r0 = (sem_r0a, sem_r0b)
    sem_r1 = (sem_r1a, sem_r1b)
    sem_o = (sem_oa, sem_ob)

    pltpu.sync_copy(idx0_hbm.at[wid], idx0_v)
    pltpu.sync_copy(idx1_hbm.at[wid], idx1_v)

    def start_gather(ci, p):
        pltpu.async_copy(
            emb_hbm.at[idx0_v.at[ci]], rows0_v.at[p], sem_r0[p])
        pltpu.async_copy(
            emb_hbm.at[idx1_v.at[ci]], rows1_v.at[p], sem_r1[p])

    def wait_gather(p):
        pltpu.make_async_copy(emb_hbm.at[idx0_v.at[0]],
                              rows0_v.at[p], sem_r0[p]).wait()
        pltpu.make_async_copy(emb_hbm.at[idx1_v.at[0]],
                              rows1_v.at[p], sem_r1[p]).wait()

    def compute(ci, p):
        rows0 = rows0_v.at[p]
        rows1 = rows1_v.at[p]

        def group_body(g, _):
            res = jnp.zeros((L,), jnp.float32)
            for j in range(L):
                e = g * L + j
                accs = [jnp.zeros((L,), jnp.float32) for _ in range(4)]
                for k in range(D // L):
                    a = rows0[e, pl.ds(k * L, L)]
                    b = rows1[e, pl.ds(k * L, L)]
                    accs[k % 4] = accs[k % 4] + a * b
                acc = (accs[0] + accs[1]) + (accs[2] + accs[3])
                res = jnp.where(lane == j, jnp.sum(acc), res)
            out_v[p, pl.ds(g * L, L)] = res
            return 0

        lax.fori_loop(0, NGROUP, group_body, 0)
        pltpu.async_copy(out_v.at[p], out_hbm.at[pl.ds(base + ci * C, C)],
                         sem_o[p])

    start_gather(0, 0)

    def chunk_pair(i, _):
        c0 = i * 2
        start_gather(c0 + 1, 1)
        wait_gather(0)
        compute(c0, 0)
        nxt = jnp.minimum(c0 + 2, NCHUNK - 1)
        start_gather(nxt, 0)
        wait_gather(1)
        compute(c0 + 1, 1)
        return 0

    def chunk_pair_guarded(i, _):
        @pl.when(i > 0)
        def _():
            pltpu.make_async_copy(out_v.at[0], out_hbm.at[pl.ds(base, C)],
                                  sem_o[0]).wait()
            pltpu.make_async_copy(out_v.at[1], out_hbm.at[pl.ds(base, C)],
                                  sem_o[1]).wait()
        chunk_pair(i, None)
        return 0

    lax.fori_loop(0, NCHUNK // 2, chunk_pair_guarded, 0)
    pltpu.make_async_copy(out_v.at[0], out_hbm.at[pl.ds(base, C)],
                          sem_o[0]).wait()
    pltpu.make_async_copy(out_v.at[1], out_hbm.at[pl.ds(base, C)],
                          sem_o[1]).wait()
    wait_gather(0)
    compute(NCHUNK - 1, 0)
    pltpu.make_async_copy(out_v.at[0], out_hbm.at[pl.ds(base, C)],
                          sem_o[0]).wait()


def kernel(node_embeddings, edge_index):
    idx = edge_index.astype(jnp.int32).reshape(2, NW, NCHUNK, C)
    return _sc_dot_kernel(node_embeddings, idx[0], idx[1])


# speedup vs baseline: 9.3657x; 1.7243x over previous
import functools

import jax
import jax.numpy as jnp
from jax import lax
from jax.experimental import pallas as pl
from jax.experimental.pallas import tpu as pltpu, tpu_sc as plsc

NC = 2
NS = 16
L = 16
NW = NC * NS

E = 320000
D = 128
W = D // 2
PER_W = E // NW
C = 80
NCHUNK = PER_W // C
NGROUP = C // L

_mesh = plsc.VectorSubcoreMesh(
    core_axis_name="c", subcore_axis_name="s", num_cores=NC, num_subcores=NS
)


@functools.partial(
    pl.kernel,
    out_type=jax.ShapeDtypeStruct((E,), jnp.float32),
    mesh=_mesh,
    compiler_params=pltpu.CompilerParams(
        needs_layout_passes=False, use_tc_tiling_on_sc=False
    ),
    scratch_types=[
        pltpu.VMEM((NCHUNK, C), jnp.int32),
        pltpu.VMEM((NCHUNK, C), jnp.int32),
        pltpu.VMEM((2, C, W), jnp.int32),
        pltpu.VMEM((2, C, W), jnp.int32),
        pltpu.VMEM((2, C), jnp.float32),
        pltpu.SemaphoreType.DMA,
        pltpu.SemaphoreType.DMA,
        pltpu.SemaphoreType.DMA,
        pltpu.SemaphoreType.DMA,
        pltpu.SemaphoreType.DMA,
        pltpu.SemaphoreType.DMA,
    ],
)
def _sc_dot_kernel(emb_hbm, idx0_hbm, idx1_hbm, out_hbm,
                   idx0_v, idx1_v, rows0_v, rows1_v, out_v,
                   sem_r0a, sem_r0b, sem_r1a, sem_r1b, sem_oa, sem_ob):
    wid = lax.axis_index("s") * NC + lax.axis_index("c")
    base = wid * PER_W
    lane = lax.iota(jnp.int32, L)
    hi_mask = jnp.full((L,), -65536, jnp.int32)
    sem_r0 = (sem_r0a, sem_r0b)
    sem_r1 = (sem_r1a, sem_r1b)
    sem_o = (sem_oa, sem_ob)

    pltpu.sync_copy(idx0_hbm.at[wid], idx0_v)
    pltpu.sync_copy(idx1_hbm.at[wid], idx1_v)

    def start_gather(ci, p):
        pltpu.async_copy(
            emb_hbm.at[idx0_v.at[ci]], rows0_v.at[p], sem_r0[p])
        pltpu.async_copy(
            emb_hbm.at[idx1_v.at[ci]], rows1_v.at[p], sem_r1[p])

    def wait_gather(p):
        pltpu.make_async_copy(emb_hbm.at[idx0_v.at[0]],
                              rows0_v.at[p], sem_r0[p]).wait()
        pltpu.make_async_copy(emb_hbm.at[idx1_v.at[0]],
                              rows1_v.at[p], sem_r1[p]).wait()

    def compute(ci, p):
        rows0 = rows0_v.at[p]
        rows1 = rows1_v.at[p]

        def group_body(g, _):
            res = jnp.zeros((L,), jnp.float32)
            for j in range(L):
                e = g * L + j
                accs = [jnp.zeros((L,), jnp.float32) for _ in range(4)]
                for k in range(W // L):
                    va = rows0[e, pl.ds(k * L, L)]
                    vb = rows1[e, pl.ds(k * L, L)]
                    a_lo = plsc.bitcast(va << 16, jnp.float32)
                    b_lo = plsc.bitcast(vb << 16, jnp.float32)
                    a_hi = plsc.bitcast(va & hi_mask, jnp.float32)
                    b_hi = plsc.bitcast(vb & hi_mask, jnp.float32)
                    accs[2 * (k % 2)] = accs[2 * (k % 2)] + a_lo * b_lo
                    accs[2 * (k % 2) + 1] = accs[2 * (k % 2) + 1] + a_hi * b_hi
                acc = (accs[0] + accs[1]) + (accs[2] + accs[3])
                res = jnp.where(lane == j, jnp.sum(acc), res)
            out_v[p, pl.ds(g * L, L)] = res
            return 0

        lax.fori_loop(0, NGROUP, group_body, 0)
        pltpu.async_copy(out_v.at[p], out_hbm.at[pl.ds(base + ci * C, C)],
                         sem_o[p])

    start_gather(0, 0)

    def chunk_pair(i, _):
        c0 = i * 2
        start_gather(c0 + 1, 1)
        wait_gather(0)
        compute(c0, 0)
        nxt = jnp.minimum(c0 + 2, NCHUNK - 1)
        start_gather(nxt, 0)
        wait_gather(1)
        compute(c0 + 1, 1)
        return 0

    def chunk_pair_guarded(i, _):
        @pl.when(i > 0)
        def _():
            pltpu.make_async_copy(out_v.at[0], out_hbm.at[pl.ds(base, C)],
                                  sem_o[0]).wait()
            pltpu.make_async_copy(out_v.at[1], out_hbm.at[pl.ds(base, C)],
                                  sem_o[1]).wait()
        chunk_pair(i, None)
        return 0

    lax.fori_loop(0, NCHUNK // 2, chunk_pair_guarded, 0)
    pltpu.make_async_copy(out_v.at[0], out_hbm.at[pl.ds(base, C)],
                          sem_o[0]).wait()
    pltpu.make_async_copy(out_v.at[1], out_hbm.at[pl.ds(base, C)],
                          sem_o[1]).wait()
    wait_gather(0)
    compute(NCHUNK - 1, 0)
    pltpu.make_async_copy(out_v.at[0], out_hbm.at[pl.ds(base, C)],
                          sem_o[0]).wait()


def kernel(node_embeddings, edge_index):
    idx = edge_index.astype(jnp.int32).reshape(2, NW, NCHUNK, C)
    emb_packed = jax.lax.bitcast_convert_type(
        node_embeddings.astype(jnp.bfloat16).reshape(-1, W, 2), jnp.int32)
    return _sc_dot_kernel(emb_packed, idx[0], idx[1])


# speedup vs baseline: 10.2972x; 1.0995x over previous
import functools

import jax
import jax.numpy as jnp
from jax import lax
from jax.experimental import pallas as pl
from jax.experimental.pallas import tpu as pltpu, tpu_sc as plsc

NC = 2
NS = 16
L = 16
NW = NC * NS

E = 320000
D = 128
W = D // 2
PER_W = E // NW
C = 80
NCHUNK = PER_W // C
NGROUP = C // L

_mesh = plsc.VectorSubcoreMesh(
    core_axis_name="c", subcore_axis_name="s", num_cores=NC, num_subcores=NS
)


@functools.partial(
    pl.kernel,
    out_type=jax.ShapeDtypeStruct((E,), jnp.float32),
    mesh=_mesh,
    compiler_params=pltpu.CompilerParams(
        needs_layout_passes=False, use_tc_tiling_on_sc=False
    ),
    scratch_types=[
        pltpu.VMEM((NCHUNK, C), jnp.int32),
        pltpu.VMEM((NCHUNK, C), jnp.int32),
        pltpu.VMEM((2, C, W), jnp.int32),
        pltpu.VMEM((2, C, W), jnp.int32),
        pltpu.VMEM((2, C), jnp.float32),
        pltpu.VMEM_SHARED((10000, W), jnp.int32),
        pltpu.SemaphoreType.DMA,
        pltpu.SemaphoreType.DMA,
        pltpu.SemaphoreType.DMA,
        pltpu.SemaphoreType.DMA,
        pltpu.SemaphoreType.DMA,
        pltpu.SemaphoreType.DMA,
        pltpu.SemaphoreType.DMA,
    ],
)
def _sc_dot_kernel(emb_hbm, idx0_hbm, idx1_hbm, out_hbm,
                   idx0_v, idx1_v, rows0_v, rows1_v, out_v, table_sp,
                   sem_t, sem_r0a, sem_r0b, sem_r1a, sem_r1b, sem_oa, sem_ob):
    wid = lax.axis_index("s") * NC + lax.axis_index("c")
    base = wid * PER_W
    lane = lax.iota(jnp.int32, L)
    hi_mask = jnp.full((L,), -65536, jnp.int32)
    sem_r0 = (sem_r0a, sem_r0b)
    sem_r1 = (sem_r1a, sem_r1b)
    sem_o = (sem_oa, sem_ob)

    sid = lax.axis_index("s")
    rows_per_sub = 10000 // NS
    pltpu.async_copy(emb_hbm.at[pl.ds(sid * rows_per_sub, rows_per_sub)],
                     table_sp.at[pl.ds(sid * rows_per_sub, rows_per_sub)],
                     sem_t).wait()
    pltpu.sync_copy(idx0_hbm.at[wid], idx0_v)
    pltpu.sync_copy(idx1_hbm.at[wid], idx1_v)
    plsc.subcore_barrier()

    def start_gather(ci, p):
        pltpu.async_copy(
            table_sp.at[idx0_v.at[ci]], rows0_v.at[p], sem_r0[p])
        pltpu.async_copy(
            table_sp.at[idx1_v.at[ci]], rows1_v.at[p], sem_r1[p])

    def wait_gather(p):
        pltpu.make_async_copy(table_sp.at[idx0_v.at[0]],
                              rows0_v.at[p], sem_r0[p]).wait()
        pltpu.make_async_copy(table_sp.at[idx1_v.at[0]],
                              rows1_v.at[p], sem_r1[p]).wait()

    def compute(ci, p):
        rows0 = rows0_v.at[p]
        rows1 = rows1_v.at[p]

        def group_body(g, _):
            res = jnp.zeros((L,), jnp.float32)
            for j in range(L):
                e = g * L + j
                accs = [jnp.zeros((L,), jnp.float32) for _ in range(4)]
                for k in range(W // L):
                    va = rows0[e, pl.ds(k * L, L)]
                    vb = rows1[e, pl.ds(k * L, L)]
                    a_lo = plsc.bitcast(va << 16, jnp.float32)
                    b_lo = plsc.bitcast(vb << 16, jnp.float32)
                    a_hi = plsc.bitcast(va & hi_mask, jnp.float32)
                    b_hi = plsc.bitcast(vb & hi_mask, jnp.float32)
                    accs[2 * (k % 2)] = accs[2 * (k % 2)] + a_lo * b_lo
                    accs[2 * (k % 2) + 1] = accs[2 * (k % 2) + 1] + a_hi * b_hi
                acc = (accs[0] + accs[1]) + (accs[2] + accs[3])
                res = jnp.where(lane == j, jnp.sum(acc), res)
            out_v[p, pl.ds(g * L, L)] = res
            return 0

        lax.fori_loop(0, NGROUP, group_body, 0)
        pltpu.async_copy(out_v.at[p], out_hbm.at[pl.ds(base + ci * C, C)],
                         sem_o[p])

    start_gather(0, 0)

    def chunk_pair(i, _):
        c0 = i * 2
        start_gather(c0 + 1, 1)
        wait_gather(0)
        compute(c0, 0)
        nxt = jnp.minimum(c0 + 2, NCHUNK - 1)
        start_gather(nxt, 0)
        wait_gather(1)
        compute(c0 + 1, 1)
        return 0

    def chunk_pair_guarded(i, _):
        @pl.when(i > 0)
        def _():
            pltpu.make_async_copy(out_v.at[0], out_hbm.at[pl.ds(base, C)],
                                  sem_o[0]).wait()
            pltpu.make_async_copy(out_v.at[1], out_hbm.at[pl.ds(base, C)],
                                  sem_o[1]).wait()
        chunk_pair(i, None)
        return 0

    lax.fori_loop(0, NCHUNK // 2, chunk_pair_guarded, 0)
    pltpu.make_async_copy(out_v.at[0], out_hbm.at[pl.ds(base, C)],
                          sem_o[0]).wait()
    pltpu.make_async_copy(out_v.at[1], out_hbm.at[pl.ds(base, C)],
                          sem_o[1]).wait()
    wait_gather(0)
    compute(NCHUNK - 1, 0)
    pltpu.make_async_copy(out_v.at[0], out_hbm.at[pl.ds(base, C)],
                          sem_o[0]).wait()


def kernel(node_embeddings, edge_index):
    idx = edge_index.astype(jnp.int32).reshape(2, NW, NCHUNK, C)
    emb_packed = jax.lax.bitcast_convert_type(
        node_embeddings.astype(jnp.bfloat16).reshape(-1, W, 2), jnp.int32)
    return _sc_dot_kernel(emb_packed, idx[0], idx[1])
